# baseline (speedup 1.0000x reference)
import functools

import jax
import jax.numpy as jnp
import numpy as np
from jax.experimental import pallas as pl

NUM_XYZ_FREQ = 10
NUM_DIR_FREQ = 4
NUM_EXPERTS = 8
HIDDEN = 256
RGB_HIDDEN = 128
TEMPERATURE = 0.166667
TILE_N = 1024
D_E = HIDDEN * NUM_EXPERTS
D_H = RGB_HIDDEN * NUM_EXPERTS
PE_LANES = 128


def _pe_matrices():
    m = np.zeros((6, PE_LANES), np.float32)
    c = np.zeros((1, PE_LANES), np.float32)
    half_pi = np.float32(np.pi / 2)
    for i in range(NUM_XYZ_FREQ):
        for ax in range(3):
            m[ax, i * 3 + ax] = 2.0 ** i
            m[ax, 30 + i * 3 + ax] = 2.0 ** i
            c[0, 30 + i * 3 + ax] = half_pi
    for i in range(NUM_DIR_FREQ):
        for ax in range(3):
            m[3 + ax, 60 + i * 3 + ax] = 2.0 ** i
            m[3 + ax, 72 + i * 3 + ax] = 2.0 ** i
            c[0, 72 + i * 3 + ax] = half_pi
    return m, c


_PE_M, _PE_C = _pe_matrices()

_S1, _S2, _S3, _S4, _S5 = (9.99984593e-01, -1.66632594e-01, 8.31238828e-03,
                           -1.93162699e-04, 2.17325696e-06)
_INV2PI = 0.15915494309189535
_RC1 = 6.28125
_RC2 = 2.0 * np.pi - 6.28125


def _cheap_sin(z):
    k = jnp.round(z * _INV2PI)
    r = (z - k * _RC1) - k * _RC2
    u = r * r
    return r * (_S1 + u * (_S2 + u * (_S3 + u * (_S4 + u * _S5))))


def _fused_kernel(x_ref, gum_ref, pe_m_ref, pe_c_ref, ws_cat_ref, b0_ref,
                  w1_ref, b1_ref, ws_all_ref, bs_all_ref, wsig_bd_ref, bsig_ref,
                  wr1a_ref, br1_all_ref, wr2bd_ref, br2_32_ref, sig32_ref,
                  oh32_ref, out_ref):
    bf = jnp.bfloat16
    mm = functools.partial(jnp.dot, preferred_element_type=jnp.float32)
    mmb = lambda a, w: jnp.dot(a, w, preferred_element_type=jnp.float32).astype(bf)
    x = x_ref[...]

    z = jax.lax.dot_general(x, pe_m_ref[...],
                            (((1,), (0,)), ((), ())),
                            precision=jax.lax.Precision.HIGHEST) + pe_c_ref[...]
    sp = _cheap_sin(z)
    x_sh = jax.lax.pad(x, jnp.float32(0.0), ((0, 0, 0), (84, PE_LANES - 90, 0)))
    s = (sp + x_sh).astype(bf)

    pre = mmb(s, ws_cat_ref[...])
    h0 = jnp.maximum(pre[:, :HIDDEN] + b0_ref[...], bf(0.0))
    vdc_all = pre[:, HIDDEN:]

    y = mmb(h0, w1_ref[...])
    y = jnp.maximum(y + b1_ref[...], bf(0.0))
    so_all = jnp.maximum(mmb(y, ws_all_ref[...]) + bs_all_ref[...],
                         bf(0.0))

    sig_lin = mm(so_all, wsig_bd_ref[...]) + bsig_ref[...]
    sigmas = jax.nn.softplus(sig_lin)

    score = jnp.log(sigmas + 1e-10) / TEMPERATURE + gum_ref[...]
    index = jnp.argmax(score, axis=-1)
    lane8 = jax.lax.broadcasted_iota(jnp.int32, score.shape, 1)
    oh8 = lane8 == index[:, None]
    sigp = jnp.sum(jnp.where(oh8, sigmas, 0.0), axis=-1, keepdims=True)

    h_parts = [mmb(so_all[:, i * HIDDEN:(i + 1) * HIDDEN], wr1a_ref[i])
               for i in range(NUM_EXPERTS)]
    h_all = jnp.maximum(jnp.concatenate(h_parts, axis=-1) + vdc_all
                        + br1_all_ref[...], bf(0.0))

    lin32 = mm(h_all, wr2bd_ref[...]) + br2_32_ref[...]
    lane32 = jax.lax.broadcasted_iota(jnp.int32, lin32.shape, 1)
    sel = jnp.where((lane32 >> 2) == index[:, None], lin32, 0.0)
    t16 = sel[:, :16] + sel[:, 16:]
    t8 = t16[:, :8] + t16[:, 8:]
    t4 = t8[:, :4] + t8[:, 4:]
    lane4 = jax.lax.broadcasted_iota(jnp.int32, t4.shape, 1)
    out_ref[...] = jnp.where(lane4 < 3, jax.nn.sigmoid(t4), sigp)


@jax.jit
def kernel(x, W0, b0, W1, b1, Ws_e, bs_e, Wsig, bsig, Wr1, br1, Wr2, br2, gumbel):
    n = x.shape[0]
    bf = jnp.bfloat16

    wr1b_all = jnp.transpose(Wr1[:, HIDDEN:, :], (1, 0, 2)).reshape(27, D_H)
    f32 = jnp.float32
    ws_xyz = jnp.concatenate([W0[3:63], jnp.zeros((24, HIDDEN), f32), W0[:3],
                              jnp.zeros((PE_LANES - 87, HIDDEN), f32)], 0)
    ws_vd = jnp.concatenate([jnp.zeros((60, D_H), f32), wr1b_all[3:27],
                             jnp.zeros((3, D_H), f32), wr1b_all[:3],
                             jnp.zeros((PE_LANES - 90, D_H), f32)], 0)
    ws_cat = jnp.concatenate([ws_xyz, ws_vd], 1).astype(bf)

    ws_all = jnp.transpose(Ws_e, (1, 0, 2)).reshape(HIDDEN, D_E).astype(bf)
    bs_all = bs_e.reshape(1, D_E).astype(bf)
    eyee = jnp.eye(NUM_EXPERTS, dtype=f32)
    wsig_bd = (eyee[:, :, None] * Wsig.reshape(1, 1, HIDDEN)
               ).transpose(0, 2, 1).reshape(D_E, NUM_EXPERTS).astype(bf)
    bsig2 = jnp.broadcast_to(bsig.reshape(1, 1), (1, NUM_EXPERTS))
    wr1a = Wr1[:, :HIDDEN, :].astype(bf)
    br1_all = br1.reshape(1, D_H).astype(bf)
    wr2p = jnp.concatenate([Wr2, jnp.zeros((NUM_EXPERTS, RGB_HIDDEN, 1), f32)], 2)
    wr2bd = (eyee[:, None, :, None] * wr2p[:, :, None, :]).reshape(D_H, 32).astype(bf)
    br2p = jnp.concatenate([br2, jnp.zeros((NUM_EXPERTS, 1), f32)], 1)
    br2_32 = br2p.reshape(1, 32)
    sig32 = jnp.kron(jnp.eye(NUM_EXPERTS, dtype=f32),
                     jnp.array([[0.0, 0.0, 0.0, 1.0]], f32))
    oh32 = jnp.kron(jnp.eye(NUM_EXPERTS, dtype=f32),
                    jnp.ones((1, 4), f32))

    grid = (n // TILE_N,)
    full = lambda s: pl.BlockSpec(s, lambda i: (0,) * len(s))
    row = lambda c: pl.BlockSpec((TILE_N, c), lambda i: (i, 0))

    out = pl.pallas_call(
        _fused_kernel,
        grid=grid,
        in_specs=[
            row(6), row(NUM_EXPERTS),
            full((6, PE_LANES)), full((1, PE_LANES)),
            full((PE_LANES, HIDDEN + D_H)), full((1, HIDDEN)),
            full((HIDDEN, HIDDEN)), full((1, HIDDEN)),
            full((HIDDEN, D_E)), full((1, D_E)), full((D_E, NUM_EXPERTS)),
            full((1, NUM_EXPERTS)),
            full((NUM_EXPERTS, HIDDEN, RGB_HIDDEN)), full((1, D_H)),
            full((D_H, 32)), full((1, 32)),
            full((NUM_EXPERTS, 32)), full((NUM_EXPERTS, 32)),
        ],
        out_specs=row(4),
        out_shape=jax.ShapeDtypeStruct((n, 4), jnp.float32),
    )(x, gumbel, jnp.asarray(_PE_M), jnp.asarray(_PE_C),
      ws_cat, b0.reshape(1, -1).astype(bf), W1.astype(bf),
      b1.reshape(1, -1).astype(bf),
      ws_all, bs_all, wsig_bd, bsig2,
      wr1a, br1_all, wr2bd, br2_32, sig32, oh32)
    return out


# speedup vs baseline: 1.0565x; 1.0565x over previous
import functools

import jax
import jax.numpy as jnp
import numpy as np
from jax.experimental import pallas as pl

NUM_XYZ_FREQ = 10
NUM_DIR_FREQ = 4
NUM_EXPERTS = 8
HIDDEN = 256
RGB_HIDDEN = 128
TEMPERATURE = 0.166667
TILE_N = 2048
D_E = HIDDEN * NUM_EXPERTS
D_H = RGB_HIDDEN * NUM_EXPERTS
PE_LANES = 128


def _pe_matrices():
    m = np.zeros((6, PE_LANES), np.float32)
    c = np.zeros((1, PE_LANES), np.float32)
    half_pi = np.float32(np.pi / 2)
    for i in range(NUM_XYZ_FREQ):
        for ax in range(3):
            m[ax, i * 3 + ax] = 2.0 ** i
            m[ax, 30 + i * 3 + ax] = 2.0 ** i
            c[0, 30 + i * 3 + ax] = half_pi
    for i in range(NUM_DIR_FREQ):
        for ax in range(3):
            m[3 + ax, 60 + i * 3 + ax] = 2.0 ** i
            m[3 + ax, 72 + i * 3 + ax] = 2.0 ** i
            c[0, 72 + i * 3 + ax] = half_pi
    return m, c


_PE_M, _PE_C = _pe_matrices()

_S1, _S2, _S3, _S4, _S5 = (9.99984593e-01, -1.66632594e-01, 8.31238828e-03,
                           -1.93162699e-04, 2.17325696e-06)
_INV2PI = 0.15915494309189535
_RC1 = 6.28125
_RC2 = 2.0 * np.pi - 6.28125


def _cheap_sin(z):
    k = jnp.round(z * _INV2PI)
    r = (z - k * _RC1) - k * _RC2
    u = r * r
    return r * (_S1 + u * (_S2 + u * (_S3 + u * (_S4 + u * _S5))))


def _fused_kernel(x_ref, gum_ref, pe_m_ref, pe_c_ref, ws_cat_ref, b0_ref,
                  w1_ref, b1_ref, ws_all_ref, bs_all_ref, wsig_t_ref, bsig_ref,
                  wr1a_ref, br1mat_ref, wr2cat_ref, br2_32_ref, out_ref):
    bf = jnp.bfloat16
    mm = functools.partial(jnp.dot, preferred_element_type=jnp.float32)
    mmb = lambda a, w: jnp.dot(a, w, preferred_element_type=jnp.float32).astype(bf)
    x = x_ref[...]

    z = jax.lax.dot_general(x, pe_m_ref[...],
                            (((1,), (0,)), ((), ())),
                            precision=jax.lax.Precision.HIGHEST) + pe_c_ref[...]
    sp = _cheap_sin(z)
    x_sh = jax.lax.pad(x, jnp.float32(0.0), ((0, 0, 0), (84, PE_LANES - 90, 0)))
    s = (sp + x_sh).astype(bf)

    pre = mmb(s, ws_cat_ref[...])
    h0 = jnp.maximum(pre[:, :HIDDEN] + b0_ref[...], bf(0.0))
    vdc_all = pre[:, HIDDEN:]

    y = mmb(h0, w1_ref[...])
    y = jnp.maximum(y + b1_ref[...], bf(0.0))
    so_all = jnp.maximum(mmb(y, ws_all_ref[...]) + bs_all_ref[...],
                         bf(0.0))

    prod = so_all.astype(jnp.float32) * wsig_t_ref[...]
    sig_lin = jnp.concatenate(
        [jnp.sum(prod[:, i * HIDDEN:(i + 1) * HIDDEN], axis=-1, keepdims=True)
         for i in range(NUM_EXPERTS)], axis=-1) + bsig_ref[...]
    sigmas = jax.nn.softplus(sig_lin)

    score = jnp.log(sigmas + 1e-10) / TEMPERATURE + gum_ref[...]
    index = jnp.argmax(score, axis=-1)
    lane8 = jax.lax.broadcasted_iota(jnp.int32, score.shape, 1)
    oh8 = lane8 == index[:, None]
    sigp = jnp.sum(jnp.where(oh8, sigmas, 0.0), axis=-1, keepdims=True)
    ohb = oh8.astype(bf)

    h_parts = [mmb(so_all[:, i * HIDDEN:(i + 1) * HIDDEN], wr1a_ref[i])
               for i in range(NUM_EXPERTS)]
    hacc = sum(h_parts[i] * ohb[:, i:i + 1] for i in range(NUM_EXPERTS))
    vdsel = sum(vdc_all[:, i * RGB_HIDDEN:(i + 1) * RGB_HIDDEN] * ohb[:, i:i + 1]
                for i in range(NUM_EXPERTS))
    br1sel = mmb(ohb, br1mat_ref[...])
    h_sel = jnp.maximum(hacc + vdsel + br1sel, bf(0.0))

    lin32 = mm(h_sel, wr2cat_ref[...]) + br2_32_ref[...]
    lane32 = jax.lax.broadcasted_iota(jnp.int32, lin32.shape, 1)
    sel = jnp.where((lane32 >> 2) == index[:, None], lin32, 0.0)
    t16 = sel[:, :16] + sel[:, 16:]
    t8 = t16[:, :8] + t16[:, 8:]
    t4 = t8[:, :4] + t8[:, 4:]
    lane4 = jax.lax.broadcasted_iota(jnp.int32, t4.shape, 1)
    out_ref[...] = jnp.where(lane4 < 3, jax.nn.sigmoid(t4), sigp)


@jax.jit
def kernel(x, W0, b0, W1, b1, Ws_e, bs_e, Wsig, bsig, Wr1, br1, Wr2, br2, gumbel):
    n = x.shape[0]
    bf = jnp.bfloat16

    wr1b_all = jnp.transpose(Wr1[:, HIDDEN:, :], (1, 0, 2)).reshape(27, D_H)
    f32 = jnp.float32
    ws_xyz = jnp.concatenate([W0[3:63], jnp.zeros((24, HIDDEN), f32), W0[:3],
                              jnp.zeros((PE_LANES - 87, HIDDEN), f32)], 0)
    ws_vd = jnp.concatenate([jnp.zeros((60, D_H), f32), wr1b_all[3:27],
                             jnp.zeros((3, D_H), f32), wr1b_all[:3],
                             jnp.zeros((PE_LANES - 90, D_H), f32)], 0)
    ws_cat = jnp.concatenate([ws_xyz, ws_vd], 1).astype(bf)

    ws_all = jnp.transpose(Ws_e, (1, 0, 2)).reshape(HIDDEN, D_E).astype(bf)
    bs_all = bs_e.reshape(1, D_E).astype(bf)
    wsig_t = jnp.tile(Wsig.reshape(1, HIDDEN), (1, NUM_EXPERTS))
    bsig2 = jnp.broadcast_to(bsig.reshape(1, 1), (1, NUM_EXPERTS))
    wr1a = Wr1[:, :HIDDEN, :].astype(bf)
    br1mat = br1.astype(bf)
    wr2p = jnp.concatenate([Wr2, jnp.zeros((NUM_EXPERTS, RGB_HIDDEN, 1), f32)], 2)
    wr2cat = jnp.transpose(wr2p, (1, 0, 2)).reshape(RGB_HIDDEN, 32).astype(bf)
    br2p = jnp.concatenate([br2, jnp.zeros((NUM_EXPERTS, 1), f32)], 1)
    br2_32 = br2p.reshape(1, 32)

    grid = (n // TILE_N,)
    full = lambda s: pl.BlockSpec(s, lambda i: (0,) * len(s))
    row = lambda c: pl.BlockSpec((TILE_N, c), lambda i: (i, 0))

    out = pl.pallas_call(
        _fused_kernel,
        grid=grid,
        in_specs=[
            row(6), row(NUM_EXPERTS),
            full((6, PE_LANES)), full((1, PE_LANES)),
            full((PE_LANES, HIDDEN + D_H)), full((1, HIDDEN)),
            full((HIDDEN, HIDDEN)), full((1, HIDDEN)),
            full((HIDDEN, D_E)), full((1, D_E)), full((1, D_E)),
            full((1, NUM_EXPERTS)),
            full((NUM_EXPERTS, HIDDEN, RGB_HIDDEN)),
            full((NUM_EXPERTS, RGB_HIDDEN)),
            full((RGB_HIDDEN, 32)), full((1, 32)),
        ],
        out_specs=row(4),
        out_shape=jax.ShapeDtypeStruct((n, 4), jnp.float32),
    )(x, gumbel, jnp.asarray(_PE_M), jnp.asarray(_PE_C),
      ws_cat, b0.reshape(1, -1).astype(bf), W1.astype(bf),
      b1.reshape(1, -1).astype(bf),
      ws_all, bs_all, wsig_t, bsig2,
      wr1a, br1mat, wr2cat, br2_32)
    return out
